# baseline (speedup 1.0000x reference)
import jax
import jax.numpy as jnp
from jax import lax
from jax.experimental import pallas as pl

_VOCAB = 1000
_EMB = 128
_BATCH = 16384
_BM = 1024
_NB = _BATCH // _BM


def _tc_fused_kernel(x_ref, t_ref, w_ref, b_ref, o_ref, e_ref):
    xl = x_ref[0]
    iota = lax.broadcasted_iota(jnp.int32, (_VOCAB, _BM), 0)
    oht = (xl == iota).astype(jnp.float32)
    emb = lax.dot_general(
        oht, t_ref[...],
        dimension_numbers=(((0,), (0,)), ((), ())),
        preferred_element_type=jnp.float32,
    )
    e_ref[...] = emb
    o_ref[...] = lax.dot_general(
        emb, w_ref[...],
        dimension_numbers=(((1,), (1,)), ((), ())),
        preferred_element_type=jnp.float32,
    ) + b_ref[0:1, :]


@jax.jit
def kernel(x, table, W, b):
    xi = x.astype(jnp.int32)
    out, emb = pl.pallas_call(
        _tc_fused_kernel,
        grid=(_NB,),
        in_specs=[
            pl.BlockSpec((1, 1, _BM), lambda i: (i, 0, 0)),
            pl.BlockSpec((_VOCAB, _EMB), lambda i: (0, 0)),
            pl.BlockSpec((_VOCAB, _EMB), lambda i: (0, 0)),
            pl.BlockSpec((1, _VOCAB), lambda i: (0, 0)),
        ],
        out_specs=[pl.BlockSpec((_BM, _VOCAB), lambda i: (i, 0)),
                   pl.BlockSpec((_BM, _EMB), lambda i: (i, 0))],
        out_shape=[jax.ShapeDtypeStruct((_BATCH, _VOCAB), jnp.float32),
                   jax.ShapeDtypeStruct((_BATCH, _EMB), jnp.float32)],
    )(xi.reshape(_NB, 1, _BM), table, W, b.reshape(1, _VOCAB))
    return out, emb


# speedup vs baseline: 1.0163x; 1.0163x over previous
import jax
import jax.numpy as jnp
from jax import lax
from jax.experimental import pallas as pl

_VOCAB = 1000
_EMB = 128
_BATCH = 16384
_BM = 2048
_NB = _BATCH // _BM


def _tc_fused_kernel(x_ref, t_ref, w_ref, b_ref, o_ref, e_ref):
    xl = x_ref[0]
    iota = lax.broadcasted_iota(jnp.int32, (_VOCAB, _BM), 0)
    oht = (xl == iota).astype(jnp.float32)
    emb = lax.dot_general(
        oht, t_ref[...],
        dimension_numbers=(((0,), (0,)), ((), ())),
        preferred_element_type=jnp.float32,
    )
    e_ref[...] = emb
    o_ref[...] = lax.dot_general(
        emb, w_ref[...],
        dimension_numbers=(((1,), (1,)), ((), ())),
        preferred_element_type=jnp.float32,
    ) + b_ref[0:1, :]


@jax.jit
def kernel(x, table, W, b):
    xi = x.astype(jnp.int32)
    out, emb = pl.pallas_call(
        _tc_fused_kernel,
        grid=(_NB,),
        in_specs=[
            pl.BlockSpec((1, 1, _BM), lambda i: (i, 0, 0)),
            pl.BlockSpec((_VOCAB, _EMB), lambda i: (0, 0)),
            pl.BlockSpec((_VOCAB, _EMB), lambda i: (0, 0)),
            pl.BlockSpec((1, _VOCAB), lambda i: (0, 0)),
        ],
        out_specs=[pl.BlockSpec((_BM, _VOCAB), lambda i: (i, 0)),
                   pl.BlockSpec((_BM, _EMB), lambda i: (i, 0))],
        out_shape=[jax.ShapeDtypeStruct((_BATCH, _VOCAB), jnp.float32),
                   jax.ShapeDtypeStruct((_BATCH, _EMB), jnp.float32)],
    )(xi.reshape(_NB, 1, _BM), table, W, b.reshape(1, _VOCAB))
    return out, emb
